# baseline (speedup 1.0000x reference)
import functools

import jax
import jax.numpy as jnp
from jax.experimental import pallas as pl
from jax.experimental.pallas import tpu as pltpu

_WEIGHT_POINT = 100.0
_WEIGHT_EDGE = 100.0


def _loss_body(p_ref, t_ref, m_ref, e_ref, y_ref, n_ref, o_ref, *, cp, ce,
               nchunks):
    rows = p_ref.shape[0] // nchunks

    def chunk(k, acc):
        sl = pl.ds(k * rows, rows)
        d = (p_ref[sl, :] - t_ref[sl, :]) * m_ref[sl, :]
        return acc + jnp.sum(d * d, dtype=jnp.float32)

    s_point = jax.lax.fori_loop(0, nchunks, chunk, jnp.float32(0.0))

    e = e_ref[...]
    y = y_ref[...]
    n = n_ref[...]
    ii = jax.lax.broadcasted_iota(jnp.int32, e.shape, 0)
    jj = jax.lax.broadcasted_iota(jnp.int32, e.shape, 1)
    nb = n[None, None]
    valid = (ii < nb) & (jj < nb)
    tgt = jj == y[:, None]
    log_p = jnp.maximum(jnp.log(e), -100.0)
    log_1mp = jnp.maximum(jnp.log(1.0 - e), -100.0)
    bce = -jnp.where(tgt, log_p, log_1mp)
    s_edge = jnp.sum(jnp.where(valid, bce, 0.0), dtype=jnp.float32)

    o_ref[0, 0] = cp * s_point + ce * s_edge


def kernel(points_pred, targets, mask, edges_pred, match_targets, npoints):
    B, C, H, W = points_pred.shape
    F = C * H * W
    M = match_targets.shape[1]
    S, L = 8, B // 8

    pt = jnp.transpose(points_pred, (1, 2, 3, 0)).reshape(F, B)
    tt = jnp.transpose(targets, (1, 2, 3, 0)).reshape(F, B)
    mt = jnp.transpose(mask, (1, 2, 3, 0)).reshape(F, B)
    e4 = jnp.transpose(edges_pred, (2, 1, 0)).reshape(M, M, S, L)
    y3 = jnp.transpose(match_targets, (1, 2, 0)).reshape(M, S, L)
    n2 = npoints.reshape(S, L)

    cp = _WEIGHT_POINT / (B * F)
    ce = _WEIGHT_EDGE / (B * M * M)
    body = functools.partial(_loss_body, cp=cp, ce=ce, nchunks=8)

    vmem = pl.BlockSpec(memory_space=pltpu.MemorySpace.VMEM)
    out = pl.pallas_call(
        body,
        in_specs=[vmem, vmem, vmem, vmem, vmem, vmem],
        out_specs=pl.BlockSpec(memory_space=pltpu.MemorySpace.SMEM),
        out_shape=jax.ShapeDtypeStruct((1, 1), jnp.float32),
    )(pt, tt, mt, e4, y3, n2)
    return out.reshape(())
